# baseline (speedup 1.0000x reference)
import jax
import jax.numpy as jnp
from jax.experimental import pallas as pl
from jax.experimental.pallas import tpu as pltpu

N = 10000
E = 640000
B = 64
D = 16
R = 114
RATIO = 0.8
H = 16
L = 3


def _head_body(ga_ref, wih_t_ref, whh_t_ref, bih_ref, bhh_ref,
               l1w_t_ref, l1b_ref, l2w_t_ref, l2b_ref, y_ref, outs_ref, gi_ref):
    gi_ref[...] = ga_ref[...] @ wih_t_ref[...] + bih_ref[...]
    whh_t = whh_t_ref[...]
    bhh = bhh_ref[...]

    def step(t, h):
        gi = gi_ref[pl.ds(t, 1), :]
        gh = h @ whh_t + bhh
        ir, iz, inn = gi[:, :H], gi[:, H:2 * H], gi[:, 2 * H:]
        hr, hz, hn = gh[:, :H], gh[:, H:2 * H], gh[:, 2 * H:]
        r = jax.nn.sigmoid(ir + hr)
        z = jax.nn.sigmoid(iz + hz)
        n = jnp.tanh(inn + r * hn)
        hnew = (1.0 - z) * n + z * h
        outs_ref[pl.ds(t, 1), :] = hnew
        return hnew

    jax.lax.fori_loop(0, B, step, jnp.zeros((1, H), jnp.float32))
    outs = outs_ref[...]
    y1 = jax.nn.relu(outs @ l1w_t_ref[...] + l1b_ref[...])
    y2 = jax.nn.relu(y1 @ l2w_t_ref[...] + l2b_ref[...])
    y_ref[...] = y2


def _head(ga, gru_wih, gru_whh, gru_bih, gru_bhh, lin1_w, lin1_b, lin2_w, lin2_b):
    return pl.pallas_call(
        _head_body,
        out_shape=jax.ShapeDtypeStruct((B, 1), jnp.float32),
        scratch_shapes=[pltpu.VMEM((B, H), jnp.float32),
                        pltpu.VMEM((B, 3 * H), jnp.float32)],
    )(ga, gru_wih.T, gru_whh.T, gru_bih[None, :], gru_bhh[None, :],
      lin1_w.T, lin1_b[None, :], lin2_w.T, lin2_b[None, :])


def _rgat(h, src, dst, etype, e_alive, W, q, k, b):
    hw = jnp.einsum('nd,rdo->nro', h, W)
    xi = hw[dst, etype]
    xj = hw[src, etype]
    alpha = jax.nn.leaky_relu(xi @ q + xj @ k, 0.2)
    alpha = jnp.where(e_alive[:, None], alpha, -jnp.inf)
    amax = jax.ops.segment_max(alpha, dst, num_segments=N)
    amax = jnp.where(jnp.isfinite(amax), amax, 0.0)
    ex = jnp.where(e_alive[:, None], jnp.exp(alpha - amax[dst]), 0.0)
    den = jax.ops.segment_sum(ex, dst, num_segments=N)
    den = jnp.where(den > 0, den, 1.0)
    out = jax.ops.segment_sum((ex / den[dst]) * xj, dst, num_segments=N)
    return out + b


def _pool(h, w, seq, n_alive, src, dst, e_alive, starts):
    score = jnp.tanh((h @ w) / jnp.linalg.norm(w))
    mscore = jnp.where(n_alive, score, -jnp.inf)
    order = jnp.lexsort((-mscore, seq))
    ranks_sorted = (jnp.arange(N) - starts[seq[order]]).astype(jnp.int32)
    rank = jnp.zeros((N,), dtype=jnp.int32).at[order].set(ranks_sorted)
    cnt = jax.ops.segment_sum(n_alive.astype(jnp.float32), seq, num_segments=B)
    kg = jnp.ceil(RATIO * cnt).astype(jnp.int32)
    keep = n_alive & (rank < kg[seq])
    h2 = jnp.where(keep[:, None], h * score[:, None], 0.0)
    e2 = e_alive & keep[src] & keep[dst]
    return h2, keep, e2


def kernel(x, edge_index, edge_attr, seq, emb, rgat_W, rgat_q, rgat_k, rgat_b,
           pool_w, gru_wih, gru_whh, gru_bih, gru_bhh, lin1_w, lin1_b, lin2_w, lin2_b):
    src = edge_index[0]
    dst = edge_index[1]
    etype = edge_attr[:, 0]
    h = emb[x[:, 0]]
    n_alive = jnp.ones((N,), dtype=bool)
    e_alive = jnp.ones((E,), dtype=bool)
    counts_all = jnp.bincount(seq, length=B)
    starts = jnp.concatenate([jnp.zeros((1,), counts_all.dtype), jnp.cumsum(counts_all)[:-1]])
    feats = []
    for l in range(L):
        h = jax.nn.relu(_rgat(h, src, dst, etype, e_alive, rgat_W[l], rgat_q[l], rgat_k[l], rgat_b[l]))
        cnt = jax.ops.segment_sum(n_alive.astype(jnp.float32), seq, num_segments=B)
        mean_g = jax.ops.segment_sum(jnp.where(n_alive[:, None], h, 0.0), seq, num_segments=B) / jnp.maximum(cnt, 1.0)[:, None]
        mx = jax.ops.segment_max(jnp.where(n_alive[:, None], h, -jnp.inf), seq, num_segments=B)
        max_g = jnp.where(jnp.isfinite(mx), mx, 0.0)
        feats.append(mean_g)
        feats.append(max_g)
        if l < L - 1:
            h, n_alive, e_alive = _pool(h, pool_w[l], seq, n_alive, src, dst, e_alive, starts)
    ga = jnp.concatenate(feats, axis=1)
    return _head(ga, gru_wih, gru_whh, gru_bih, gru_bhh, lin1_w, lin1_b, lin2_w, lin2_b)


# speedup vs baseline: 9.8458x; 9.8458x over previous
import functools

import jax
import jax.numpy as jnp
from jax import lax
from jax.experimental import pallas as pl
from jax.experimental.pallas import tpu as pltpu
from jax.experimental.pallas import tpu_sc as plsc

N = 10000
E = 640000
B = 64
D = 16
R = 114
RATIO = 0.8
H = 16
L = 3

NC = 2
NS = 16
NW = NC * NS
KB = 128
NBT = 157
EWP = NBT * KB
EPAD = NW * EWP
NEG = -3.0e8


def _sc_edge_body(hw_hbm, idxs_hbm, idxd_hbm, src_hbm, dst_hbm,
                  flag_hbm, qk_hbm, out_hbm,
                  flag_v, idxs_b, idxd_b, dstb_i, srcf_b, dstf_b,
                  rows_s, rows_d, out_v, qkv, zbuf,
                  accum_sh, sem1, sem2):
    c = lax.axis_index("c")
    s = lax.axis_index("s")
    wid = c * NS + s

    pltpu.sync_copy(flag_hbm, flag_v)
    pltpu.sync_copy(qk_hbm, qkv)

    zeros16 = jnp.zeros((16,), jnp.float32)

    def zz(i, _):
        zbuf[i, pl.ds(0, 16)] = zeros16
        zbuf[i, pl.ds(16, 16)] = zeros16
        return 0
    lax.fori_loop(0, 125, zz, 0)
    for i in range(5):
        pltpu.sync_copy(zbuf, accum_sh.at[pl.ds(s * 625 + i * 125, 125)])
    plsc.subcore_barrier()

    ebase0 = wid * EWP

    def batch(b, _):
        pltpu.sync_copy(idxs_hbm.at[wid, b], idxs_b)
        pltpu.sync_copy(idxd_hbm.at[wid, b], idxd_b)
        pltpu.sync_copy(dst_hbm.at[wid, b], dstb_i)
        pltpu.sync_copy(src_hbm.at[wid, b], srcf_b.at[pl.ds(0, KB)])
        pltpu.sync_copy(dst_hbm.at[wid, b], dstf_b.at[pl.ds(0, KB)])
        cp1 = pltpu.async_copy(hw_hbm.at[idxs_b], rows_s, sem1)
        cp2 = pltpu.async_copy(hw_hbm.at[idxd_b], rows_d, sem2)
        cp1.wait()
        cp2.wait()

        def edge(e, _2):
            iot = lax.iota(jnp.int32, 16)
            row_s = rows_s[e, :]
            row_d = rows_d[e, :]
            xv = row_d * qkv[0, :] + row_s * qkv[1, :]
            for sh in (8, 4, 2, 1):
                xv = xv + xv.at[iot ^ sh].get(mode="promise_in_bounds")
            alv = jnp.maximum(xv, 0.2 * xv)
            sid = srcf_b[pl.ds(e, 16)][0]
            did = dstf_b[pl.ds(e, 16)][0]
            fs = flag_v[pl.ds(sid, 16)][0]
            fd = flag_v[pl.ds(did, 16)][0]
            eg = ebase0 + b * KB + e
            alv = jnp.where(eg < E, alv + jnp.full((16,), fs + fd),
                            jnp.full((16,), NEG))
            wv = jnp.exp(alv)
            out_v[e, pl.ds(0, 16)] = row_s * wv
            out_v[e, pl.ds(16, 16)] = qkv[2, :] * wv
            return 0

        lax.fori_loop(0, KB, edge, 0)
        pltpu.sync_copy(out_v, accum_sh.at[dstb_i], add=True)
        return 0

    lax.fori_loop(0, NBT, batch, 0)
    plsc.subcore_barrier()

    @pl.when(s == 0)
    def _():
        pltpu.sync_copy(accum_sh, out_hbm.at[c])


@functools.partial(jax.jit, static_argnums=())
def _sc_edge(hw_flat, idxs3, idxd3, src3, dst3, flag, qk):
    mesh = plsc.VectorSubcoreMesh(core_axis_name="c", subcore_axis_name="s")
    f = pl.kernel(
        _sc_edge_body,
        mesh=mesh,
        compiler_params=pltpu.CompilerParams(use_tc_tiling_on_sc=False),
        out_type=jax.ShapeDtypeStruct((NC, N, 32), jnp.float32),
        scratch_types=[
            pltpu.VMEM((N + 16,), jnp.float32),
            pltpu.VMEM((KB,), jnp.int32),
            pltpu.VMEM((KB,), jnp.int32),
            pltpu.VMEM((KB,), jnp.int32),
            pltpu.VMEM((KB + 16,), jnp.int32),
            pltpu.VMEM((KB + 16,), jnp.int32),
            pltpu.VMEM((KB, D), jnp.float32),
            pltpu.VMEM((KB, D), jnp.float32),
            pltpu.VMEM((KB, 32), jnp.float32),
            pltpu.VMEM((4, 16), jnp.float32),
            pltpu.VMEM((125, 32), jnp.float32),
            pltpu.VMEM_SHARED((N, 32), jnp.float32),
            pltpu.SemaphoreType.DMA,
            pltpu.SemaphoreType.DMA,
        ],
    )
    return f(hw_flat, idxs3, idxd3, src3, dst3, flag, qk)


def _pad3(a):
    a = jnp.concatenate([a, jnp.zeros((EPAD - E,), a.dtype)])
    return a.reshape(NW, NBT, KB)


def _rgat_sc(h, idxs3, idxd3, src3, dst3, flag, W, q, k, b):
    hw = jnp.einsum('nd,rdo->nro', h, W).reshape(N * R, D)
    e0 = jnp.zeros((D,), jnp.float32).at[0].set(1.0)
    qk = jnp.stack([q[:, 0], k[:, 0], e0, jnp.zeros((D,), jnp.float32)])
    flagp = jnp.concatenate([flag, jnp.zeros((16,), jnp.float32)])
    acc2 = _sc_edge(hw, idxs3, idxd3, src3, dst3, flagp, qk)
    acc = acc2[0] + acc2[1]
    num = acc[:, :D]
    den = acc[:, D]
    den = jnp.where(den > 0, den, 1.0)
    return num / den[:, None] + b


def _head_body(ga_ref, wih_t_ref, whh_t_ref, bih_ref, bhh_ref,
               l1w_t_ref, l1b_ref, l2w_t_ref, l2b_ref, y_ref, outs_ref, gi_ref):
    gi_ref[...] = ga_ref[...] @ wih_t_ref[...] + bih_ref[...]
    whh_t = whh_t_ref[...]
    bhh = bhh_ref[...]

    def step(t, h):
        gi = gi_ref[pl.ds(t, 1), :]
        gh = h @ whh_t + bhh
        ir, iz, inn = gi[:, :H], gi[:, H:2 * H], gi[:, 2 * H:]
        hr, hz, hn = gh[:, :H], gh[:, H:2 * H], gh[:, 2 * H:]
        r = jax.nn.sigmoid(ir + hr)
        z = jax.nn.sigmoid(iz + hz)
        n = jnp.tanh(inn + r * hn)
        hnew = (1.0 - z) * n + z * h
        outs_ref[pl.ds(t, 1), :] = hnew
        return hnew

    jax.lax.fori_loop(0, B, step, jnp.zeros((1, H), jnp.float32))
    outs = outs_ref[...]
    y1 = jax.nn.relu(outs @ l1w_t_ref[...] + l1b_ref[...])
    y2 = jax.nn.relu(y1 @ l2w_t_ref[...] + l2b_ref[...])
    y_ref[...] = y2


def _head(ga, gru_wih, gru_whh, gru_bih, gru_bhh, lin1_w, lin1_b, lin2_w, lin2_b):
    return pl.pallas_call(
        _head_body,
        out_shape=jax.ShapeDtypeStruct((B, 1), jnp.float32),
        scratch_shapes=[pltpu.VMEM((B, H), jnp.float32),
                        pltpu.VMEM((B, 3 * H), jnp.float32)],
    )(ga, gru_wih.T, gru_whh.T, gru_bih[None, :], gru_bhh[None, :],
      lin1_w.T, lin1_b[None, :], lin2_w.T, lin2_b[None, :])


def _pool(h, w, seq, n_alive, src, dst, starts):
    score = jnp.tanh((h @ w) / jnp.linalg.norm(w))
    mscore = jnp.where(n_alive, score, -jnp.inf)
    order = jnp.lexsort((-mscore, seq))
    ranks_sorted = (jnp.arange(N) - starts[seq[order]]).astype(jnp.int32)
    rank = jnp.zeros((N,), dtype=jnp.int32).at[order].set(ranks_sorted)
    cnt = jax.ops.segment_sum(n_alive.astype(jnp.float32), seq, num_segments=B)
    kg = jnp.ceil(RATIO * cnt).astype(jnp.int32)
    keep = n_alive & (rank < kg[seq])
    h2 = jnp.where(keep[:, None], h * score[:, None], 0.0)
    return h2, keep


def kernel(x, edge_index, edge_attr, seq, emb, rgat_W, rgat_q, rgat_k, rgat_b,
           pool_w, gru_wih, gru_whh, gru_bih, gru_bhh, lin1_w, lin1_b, lin2_w, lin2_b):
    src = edge_index[0]
    dst = edge_index[1]
    etype = edge_attr[:, 0]
    idxs3 = _pad3(src * R + etype)
    idxd3 = _pad3(dst * R + etype)
    src3 = _pad3(src)
    dst3 = _pad3(dst)
    h = emb[x[:, 0]]
    n_alive = jnp.ones((N,), dtype=bool)
    flag = jnp.zeros((N,), jnp.float32)
    counts_all = jnp.bincount(seq, length=B)
    starts = jnp.concatenate([jnp.zeros((1,), counts_all.dtype), jnp.cumsum(counts_all)[:-1]])
    feats = []
    for l in range(L):
        h = jax.nn.relu(_rgat_sc(h, idxs3, idxd3, src3, dst3, flag,
                                 rgat_W[l], rgat_q[l], rgat_k[l], rgat_b[l]))
        cnt = jax.ops.segment_sum(n_alive.astype(jnp.float32), seq, num_segments=B)
        mean_g = jax.ops.segment_sum(jnp.where(n_alive[:, None], h, 0.0), seq, num_segments=B) / jnp.maximum(cnt, 1.0)[:, None]
        mx = jax.ops.segment_max(jnp.where(n_alive[:, None], h, -jnp.inf), seq, num_segments=B)
        max_g = jnp.where(jnp.isfinite(mx), mx, 0.0)
        feats.append(mean_g)
        feats.append(max_g)
        if l < L - 1:
            h, keep = _pool(h, pool_w[l], seq, n_alive, src, dst, starts)
            n_alive = keep
            flag = jnp.where(keep, 0.0, NEG).astype(jnp.float32)
    ga = jnp.concatenate(feats, axis=1)
    return _head(ga, gru_wih, gru_whh, gru_bih, gru_bhh, lin1_w, lin1_b, lin2_w, lin2_b)


# speedup vs baseline: 10.9695x; 1.1141x over previous
import functools

import jax
import jax.numpy as jnp
from jax import lax
from jax.experimental import pallas as pl
from jax.experimental.pallas import tpu as pltpu
from jax.experimental.pallas import tpu_sc as plsc

N = 10000
E = 640000
B = 64
D = 16
R = 114
RATIO = 0.8
H = 16
L = 3

NC = 2
NS = 16
NW = NC * NS
KB = 128
NBT = 157
EWP = NBT * KB
EPAD = NW * EWP
NEG = -3.0e8


def _sc_edge_body(hw_hbm, pack_hbm, flag_hbm, qk_hbm, out_hbm,
                  flag_v, pack_b, rows_s, rows_d, out_v, qkv, zbuf,
                  accum_sh, sem1, sem2):
    c = lax.axis_index("c")
    s = lax.axis_index("s")
    wid = c * NS + s

    pltpu.sync_copy(flag_hbm, flag_v)
    pltpu.sync_copy(qk_hbm, qkv)

    zeros16 = jnp.zeros((16,), jnp.float32)

    def zz(i, _):
        zbuf[i, pl.ds(0, 16)] = zeros16
        zbuf[i, pl.ds(16, 16)] = zeros16
        return 0
    lax.fori_loop(0, 125, zz, 0)
    for i in range(5):
        pltpu.sync_copy(zbuf, accum_sh.at[pl.ds(s * 625 + i * 125, 125)])
    plsc.subcore_barrier()

    ebase0 = wid * EWP

    def batch(b, _):
        pltpu.sync_copy(pack_hbm.at[wid, b], pack_b)
        cp1 = pltpu.async_copy(hw_hbm.at[pack_b.at[0]], rows_s, sem1)
        cp2 = pltpu.async_copy(hw_hbm.at[pack_b.at[1]], rows_d, sem2)
        cp1.wait()
        cp2.wait()

        def edge(e, _2):
            iot = lax.iota(jnp.int32, 16)
            row_s = rows_s[e, :]
            row_d = rows_d[e, :]
            xv = row_d * qkv[0, :] + row_s * qkv[1, :]
            for sh in (8, 4, 2, 1):
                xv = xv + xv.at[iot ^ sh].get(mode="promise_in_bounds")
            alv = jnp.maximum(xv, 0.2 * xv)
            sid = pack_b[3, pl.ds(e, 16)][0]
            did = pack_b[4, pl.ds(e, 16)][0]
            fs = flag_v[pl.ds(sid, 16)][0]
            fd = flag_v[pl.ds(did, 16)][0]
            eg = ebase0 + b * KB + e
            alv = jnp.where(eg < E, alv + jnp.full((16,), fs + fd),
                            jnp.full((16,), NEG))
            wv = jnp.exp(alv)
            out_v[e, pl.ds(0, 16)] = row_s * wv
            out_v[e, pl.ds(16, 16)] = qkv[2, :] * wv
            return 0

        def edge2(i, _2):
            edge(2 * i, 0)
            edge(2 * i + 1, 0)
            return 0

        lax.fori_loop(0, KB // 2, edge2, 0)
        pltpu.sync_copy(out_v, accum_sh.at[pack_b.at[2]], add=True)
        return 0

    lax.fori_loop(0, NBT, batch, 0)
    plsc.subcore_barrier()

    @pl.when(s == 0)
    def _():
        pltpu.sync_copy(accum_sh, out_hbm.at[c])


@functools.partial(jax.jit, static_argnums=())
def _sc_edge(hw_flat, pack4, flag, qk):
    mesh = plsc.VectorSubcoreMesh(core_axis_name="c", subcore_axis_name="s")
    f = pl.kernel(
        _sc_edge_body,
        mesh=mesh,
        compiler_params=pltpu.CompilerParams(use_tc_tiling_on_sc=False),
        out_type=jax.ShapeDtypeStruct((NC, N, 32), jnp.float32),
        scratch_types=[
            pltpu.VMEM((N + 16,), jnp.float32),
            pltpu.VMEM((6, KB), jnp.int32),
            pltpu.VMEM((KB, D), jnp.float32),
            pltpu.VMEM((KB, D), jnp.float32),
            pltpu.VMEM((KB, 32), jnp.float32),
            pltpu.VMEM((4, 16), jnp.float32),
            pltpu.VMEM((125, 32), jnp.float32),
            pltpu.VMEM_SHARED((N, 32), jnp.float32),
            pltpu.SemaphoreType.DMA,
            pltpu.SemaphoreType.DMA,
        ],
    )
    return f(hw_flat, pack4, flag, qk)


def _pad3(a):
    a = jnp.concatenate([a, jnp.zeros((EPAD - E,), a.dtype)])
    return a.reshape(NW, NBT, KB)


def _rgat_sc(h, pack4, flag, W, q, k, b):
    hw = jnp.einsum('nd,rdo->nro', h, W).reshape(N * R, D)
    e0 = jnp.zeros((D,), jnp.float32).at[0].set(1.0)
    qk = jnp.stack([q[:, 0], k[:, 0], e0, jnp.zeros((D,), jnp.float32)])
    flagp = jnp.concatenate([flag, jnp.zeros((16,), jnp.float32)])
    acc2 = _sc_edge(hw, pack4, flagp, qk)
    acc = acc2[0] + acc2[1]
    num = acc[:, :D]
    den = acc[:, D]
    den = jnp.where(den > 0, den, 1.0)
    return num / den[:, None] + b


def _head_body(ga_ref, wih_t_ref, whh_t_ref, bih_ref, bhh_ref,
               l1w_t_ref, l1b_ref, l2w_t_ref, l2b_ref, y_ref, outs_ref, gi_ref):
    gi_ref[...] = ga_ref[...] @ wih_t_ref[...] + bih_ref[...]
    whh_t = whh_t_ref[...]
    bhh = bhh_ref[...]

    def step(t, h):
        gi = gi_ref[pl.ds(t, 1), :]
        gh = h @ whh_t + bhh
        ir, iz, inn = gi[:, :H], gi[:, H:2 * H], gi[:, 2 * H:]
        hr, hz, hn = gh[:, :H], gh[:, H:2 * H], gh[:, 2 * H:]
        r = jax.nn.sigmoid(ir + hr)
        z = jax.nn.sigmoid(iz + hz)
        n = jnp.tanh(inn + r * hn)
        hnew = (1.0 - z) * n + z * h
        outs_ref[pl.ds(t, 1), :] = hnew
        return hnew

    jax.lax.fori_loop(0, B, step, jnp.zeros((1, H), jnp.float32))
    outs = outs_ref[...]
    y1 = jax.nn.relu(outs @ l1w_t_ref[...] + l1b_ref[...])
    y2 = jax.nn.relu(y1 @ l2w_t_ref[...] + l2b_ref[...])
    y_ref[...] = y2


def _head(ga, gru_wih, gru_whh, gru_bih, gru_bhh, lin1_w, lin1_b, lin2_w, lin2_b):
    return pl.pallas_call(
        _head_body,
        out_shape=jax.ShapeDtypeStruct((B, 1), jnp.float32),
        scratch_shapes=[pltpu.VMEM((B, H), jnp.float32),
                        pltpu.VMEM((B, 3 * H), jnp.float32)],
    )(ga, gru_wih.T, gru_whh.T, gru_bih[None, :], gru_bhh[None, :],
      lin1_w.T, lin1_b[None, :], lin2_w.T, lin2_b[None, :])


def _pool(h, w, seq, n_alive, src, dst, starts):
    score = jnp.tanh((h @ w) / jnp.linalg.norm(w))
    mscore = jnp.where(n_alive, score, -jnp.inf)
    order = jnp.lexsort((-mscore, seq))
    ranks_sorted = (jnp.arange(N) - starts[seq[order]]).astype(jnp.int32)
    rank = jnp.zeros((N,), dtype=jnp.int32).at[order].set(ranks_sorted)
    cnt = jnp.sum((seq[None, :] == jnp.arange(B, dtype=seq.dtype)[:, None])
                  & n_alive[None, :], axis=1).astype(jnp.float32)
    kg = jnp.ceil(RATIO * cnt).astype(jnp.int32)
    keep = n_alive & (rank < kg[seq])
    h2 = jnp.where(keep[:, None], h * score[:, None], 0.0)
    return h2, keep


def kernel(x, edge_index, edge_attr, seq, emb, rgat_W, rgat_q, rgat_k, rgat_b,
           pool_w, gru_wih, gru_whh, gru_bih, gru_bhh, lin1_w, lin1_b, lin2_w, lin2_b):
    src = edge_index[0]
    dst = edge_index[1]
    etype = edge_attr[:, 0]
    idxs3 = _pad3(src * R + etype)
    idxd3 = _pad3(dst * R + etype)
    src3 = _pad3(src)
    dst3 = _pad3(dst)
    pack4 = jnp.stack([idxs3, idxd3, dst3, src3, dst3, dst3], axis=2)
    h = emb[x[:, 0]]
    n_alive = jnp.ones((N,), dtype=bool)
    flag = jnp.zeros((N,), jnp.float32)
    counts_all = jnp.bincount(seq, length=B)
    starts = jnp.concatenate([jnp.zeros((1,), counts_all.dtype), jnp.cumsum(counts_all)[:-1]])
    feats = []
    for l in range(L):
        h = jax.nn.relu(_rgat_sc(h, pack4, flag,
                                 rgat_W[l], rgat_q[l], rgat_k[l], rgat_b[l]))
        am = (seq[None, :] == jnp.arange(B, dtype=seq.dtype)[:, None]) & n_alive[None, :]
        amf = am.astype(jnp.float32)
        cnt = jnp.sum(amf, axis=1)
        mean_g = (amf @ h) / jnp.maximum(cnt, 1.0)[:, None]
        mx = jnp.max(jnp.where(am[:, :, None], h[None], -jnp.inf), axis=1)
        max_g = jnp.where(jnp.isfinite(mx), mx, 0.0)
        feats.append(mean_g)
        feats.append(max_g)
        if l < L - 1:
            h, keep = _pool(h, pool_w[l], seq, n_alive, src, dst, starts)
            n_alive = keep
            flag = jnp.where(keep, 0.0, NEG).astype(jnp.float32)
    ga = jnp.concatenate(feats, axis=1)
    return _head(ga, gru_wih, gru_whh, gru_bih, gru_bhh, lin1_w, lin1_b, lin2_w, lin2_b)


# speedup vs baseline: 11.2709x; 1.0275x over previous
import functools

import jax
import jax.numpy as jnp
from jax import lax
from jax.experimental import pallas as pl
from jax.experimental.pallas import tpu as pltpu
from jax.experimental.pallas import tpu_sc as plsc

N = 10000
E = 640000
B = 64
D = 16
R = 114
RATIO = 0.8
H = 16
L = 3

NC = 2
NS = 16
NW = NC * NS
KB = 128
NBT = 157
EWP = NBT * KB
EPAD = NW * EWP
NEG = -3.0e8


def _sc_edge_body(hw_hbm, pack_hbm, flag_hbm, qk_hbm, out_hbm,
                  flag_v, pack_b0, pack_b1, rows_s0, rows_s1, rows_d0, rows_d1,
                  out_v0, out_v1, sidx0, sidx1, qkv, zbuf, accum_sh,
                  gsem0a, gsem0b, gsem1a, gsem1b, psem0, psem1, ssem0, ssem1):
    c = lax.axis_index("c")
    s = lax.axis_index("s")
    wid = c * NS + s

    pltpu.sync_copy(flag_hbm, flag_v)
    pltpu.sync_copy(qk_hbm, qkv)

    zeros16 = jnp.zeros((16,), jnp.float32)

    def zz(i, _):
        zbuf[i, pl.ds(0, 16)] = zeros16
        zbuf[i, pl.ds(16, 16)] = zeros16
        return 0
    lax.fori_loop(0, 125, zz, 0)
    for i in range(5):
        pltpu.sync_copy(zbuf, accum_sh.at[pl.ds(s * 625 + i * 125, 125)])
    plsc.subcore_barrier()

    ebase0 = wid * EWP

    def mkedge(pack_b, rows_s, rows_d, out_v, b):
        def edge(e, _2):
            iot = lax.iota(jnp.int32, 16)
            row_s = rows_s[e, :]
            row_d = rows_d[e, :]
            xv = row_d * qkv[0, :] + row_s * qkv[1, :]
            for sh in (8, 4, 2, 1):
                xv = xv + xv.at[iot ^ sh].get(mode="promise_in_bounds")
            alv = jnp.maximum(xv, 0.2 * xv)
            sid = pack_b[3, pl.ds(e, 16)][0]
            did = pack_b[4, pl.ds(e, 16)][0]
            fs = flag_v[pl.ds(sid, 16)][0]
            fd = flag_v[pl.ds(did, 16)][0]
            el = b * KB + e
            alv = jnp.where((el < EWP) & (ebase0 + el < E),
                            alv + jnp.full((16,), fs + fd),
                            jnp.full((16,), NEG))
            wv = jnp.exp(alv)
            out_v[e, pl.ds(0, 16)] = row_s * wv
            out_v[e, pl.ds(16, 16)] = qkv[2, :] * wv
            return 0

        def edge2(i, _2):
            edge(2 * i, 0)
            edge(2 * i + 1, 0)
            return 0

        lax.fori_loop(0, KB // 2, edge2, 0)

    packs = (pack_b0, pack_b1)
    sidxs = (sidx0, sidx1)
    rowss = (rows_s0, rows_s1)
    rowsd = (rows_d0, rows_d1)
    outs = (out_v0, out_v1)
    gsems = ((gsem0a, gsem0b), (gsem1a, gsem1b))
    psems = (psem0, psem1)
    ssems = (ssem0, ssem1)

    pltpu.sync_copy(pack_hbm.at[wid, 0], packs[0])
    pltpu.async_copy(hw_hbm.at[packs[0].at[0]], rowss[0], gsems[0][0])
    pltpu.async_copy(hw_hbm.at[packs[0].at[1]], rowsd[0], gsems[0][1])
    pltpu.async_copy(pack_hbm.at[wid, 1], packs[1], psems[1])

    def pair(p, _):
        for par in (0, 1):
            b = 2 * p + par
            oth = 1 - par
            pltpu.make_async_copy(hw_hbm.at[packs[par].at[0]], rowss[par], gsems[par][0]).wait()
            pltpu.make_async_copy(hw_hbm.at[packs[par].at[1]], rowsd[par], gsems[par][1]).wait()
            pltpu.make_async_copy(pack_hbm.at[wid, b + 1], packs[oth], psems[oth]).wait()
            pltpu.async_copy(hw_hbm.at[packs[oth].at[0]], rowss[oth], gsems[oth][0])
            pltpu.async_copy(hw_hbm.at[packs[oth].at[1]], rowsd[oth], gsems[oth][1])
            @pl.when(b >= 2)
            def _():
                pltpu.make_async_copy(outs[par], accum_sh.at[sidxs[par]], ssems[par]).wait()
            mkedge(packs[par], rowss[par], rowsd[par], outs[par], b)
            for j in range(KB // 16):
                sidxs[par][pl.ds(j * 16, 16)] = packs[par][2, pl.ds(j * 16, 16)]
            pltpu.async_copy(outs[par], accum_sh.at[sidxs[par]], ssems[par], add=True)
            pltpu.async_copy(pack_hbm.at[wid, b + 2], packs[par], psems[par])
        return 0

    lax.fori_loop(0, (NBT + 1) // 2, pair, 0)
    for par in (0, 1):
        pltpu.make_async_copy(outs[par], accum_sh.at[sidxs[par]], ssems[par]).wait()
    pltpu.make_async_copy(pack_hbm.at[wid, 0], packs[1], psems[1]).wait()
    pltpu.make_async_copy(hw_hbm.at[packs[0].at[0]], rowss[0], gsems[0][0]).wait()
    pltpu.make_async_copy(hw_hbm.at[packs[0].at[1]], rowsd[0], gsems[0][1]).wait()
    plsc.subcore_barrier()

    @pl.when(s == 0)
    def _():
        pltpu.sync_copy(accum_sh, out_hbm.at[c])


@functools.partial(jax.jit, static_argnums=())
def _sc_edge(hw_flat, pack4, flag, qk):
    mesh = plsc.VectorSubcoreMesh(core_axis_name="c", subcore_axis_name="s")
    f = pl.kernel(
        _sc_edge_body,
        mesh=mesh,
        compiler_params=pltpu.CompilerParams(use_tc_tiling_on_sc=False),
        out_type=jax.ShapeDtypeStruct((NC, N, 32), jnp.float32),
        scratch_types=[
            pltpu.VMEM((N + 16,), jnp.float32),
            pltpu.VMEM((6, KB), jnp.int32),
            pltpu.VMEM((6, KB), jnp.int32),
            pltpu.VMEM((KB, D), jnp.float32),
            pltpu.VMEM((KB, D), jnp.float32),
            pltpu.VMEM((KB, D), jnp.float32),
            pltpu.VMEM((KB, D), jnp.float32),
            pltpu.VMEM((KB, 32), jnp.float32),
            pltpu.VMEM((KB, 32), jnp.float32),
            pltpu.VMEM((KB,), jnp.int32),
            pltpu.VMEM((KB,), jnp.int32),
            pltpu.VMEM((4, 16), jnp.float32),
            pltpu.VMEM((125, 32), jnp.float32),
            pltpu.VMEM_SHARED((N, 32), jnp.float32),
            pltpu.SemaphoreType.DMA,
            pltpu.SemaphoreType.DMA,
            pltpu.SemaphoreType.DMA,
            pltpu.SemaphoreType.DMA,
            pltpu.SemaphoreType.DMA,
            pltpu.SemaphoreType.DMA,
            pltpu.SemaphoreType.DMA,
            pltpu.SemaphoreType.DMA,
        ],
    )
    return f(hw_flat, pack4, flag, qk)


def _pad3(a):
    a = jnp.concatenate([a, jnp.zeros((EPAD - E,), a.dtype)])
    return a.reshape(NW, NBT, KB)


def _rgat_sc(h, pack4, flag, W, q, k, b):
    hw = jnp.einsum('nd,rdo->nro', h, W).reshape(N * R, D)
    e0 = jnp.zeros((D,), jnp.float32).at[0].set(1.0)
    qk = jnp.stack([q[:, 0], k[:, 0], e0, jnp.zeros((D,), jnp.float32)])
    flagp = jnp.concatenate([flag, jnp.zeros((16,), jnp.float32)])
    acc2 = _sc_edge(hw, pack4, flagp, qk)
    acc = acc2[0] + acc2[1]
    num = acc[:, :D]
    den = acc[:, D]
    den = jnp.where(den > 0, den, 1.0)
    return num / den[:, None] + b


def _head_body(ga_ref, wih_t_ref, whh_t_ref, bih_ref, bhh_ref,
               l1w_t_ref, l1b_ref, l2w_t_ref, l2b_ref, y_ref, outs_ref, gi_ref):
    gi_ref[...] = ga_ref[...] @ wih_t_ref[...] + bih_ref[...]
    whh_t = whh_t_ref[...]
    bhh = bhh_ref[...]

    def step(t, h):
        gi = gi_ref[pl.ds(t, 1), :]
        gh = h @ whh_t + bhh
        ir, iz, inn = gi[:, :H], gi[:, H:2 * H], gi[:, 2 * H:]
        hr, hz, hn = gh[:, :H], gh[:, H:2 * H], gh[:, 2 * H:]
        r = jax.nn.sigmoid(ir + hr)
        z = jax.nn.sigmoid(iz + hz)
        n = jnp.tanh(inn + r * hn)
        hnew = (1.0 - z) * n + z * h
        outs_ref[pl.ds(t, 1), :] = hnew
        return hnew

    jax.lax.fori_loop(0, B, step, jnp.zeros((1, H), jnp.float32))
    outs = outs_ref[...]
    y1 = jax.nn.relu(outs @ l1w_t_ref[...] + l1b_ref[...])
    y2 = jax.nn.relu(y1 @ l2w_t_ref[...] + l2b_ref[...])
    y_ref[...] = y2


def _head(ga, gru_wih, gru_whh, gru_bih, gru_bhh, lin1_w, lin1_b, lin2_w, lin2_b):
    return pl.pallas_call(
        _head_body,
        out_shape=jax.ShapeDtypeStruct((B, 1), jnp.float32),
        scratch_shapes=[pltpu.VMEM((B, H), jnp.float32),
                        pltpu.VMEM((B, 3 * H), jnp.float32)],
    )(ga, gru_wih.T, gru_whh.T, gru_bih[None, :], gru_bhh[None, :],
      lin1_w.T, lin1_b[None, :], lin2_w.T, lin2_b[None, :])


def _pool(h, w, seq, n_alive, src, dst, starts):
    score = jnp.tanh((h @ w) / jnp.linalg.norm(w))
    mscore = jnp.where(n_alive, score, -jnp.inf)
    order = jnp.lexsort((-mscore, seq))
    ranks_sorted = (jnp.arange(N) - starts[seq[order]]).astype(jnp.int32)
    rank = jnp.zeros((N,), dtype=jnp.int32).at[order].set(ranks_sorted)
    cnt = jnp.sum((seq[None, :] == jnp.arange(B, dtype=seq.dtype)[:, None])
                  & n_alive[None, :], axis=1).astype(jnp.float32)
    kg = jnp.ceil(RATIO * cnt).astype(jnp.int32)
    keep = n_alive & (rank < kg[seq])
    h2 = jnp.where(keep[:, None], h * score[:, None], 0.0)
    return h2, keep


def kernel(x, edge_index, edge_attr, seq, emb, rgat_W, rgat_q, rgat_k, rgat_b,
           pool_w, gru_wih, gru_whh, gru_bih, gru_bhh, lin1_w, lin1_b, lin2_w, lin2_b):
    src = edge_index[0]
    dst = edge_index[1]
    etype = edge_attr[:, 0]
    idxs3 = _pad3(src * R + etype)
    idxd3 = _pad3(dst * R + etype)
    src3 = _pad3(src)
    dst3 = _pad3(dst)
    pack4 = jnp.stack([idxs3, idxd3, dst3, src3, dst3, dst3], axis=2)
    pack4 = jnp.pad(pack4, ((0, 0), (0, 3), (0, 0), (0, 0)))
    h = emb[x[:, 0]]
    n_alive = jnp.ones((N,), dtype=bool)
    flag = jnp.zeros((N,), jnp.float32)
    counts_all = jnp.bincount(seq, length=B)
    starts = jnp.concatenate([jnp.zeros((1,), counts_all.dtype), jnp.cumsum(counts_all)[:-1]])
    feats = []
    for l in range(L):
        h = jax.nn.relu(_rgat_sc(h, pack4, flag,
                                 rgat_W[l], rgat_q[l], rgat_k[l], rgat_b[l]))
        am = (seq[None, :] == jnp.arange(B, dtype=seq.dtype)[:, None]) & n_alive[None, :]
        amf = am.astype(jnp.float32)
        cnt = jnp.sum(amf, axis=1)
        mean_g = (amf @ h) / jnp.maximum(cnt, 1.0)[:, None]
        mx = jnp.max(jnp.where(am[:, :, None], h[None], -jnp.inf), axis=1)
        max_g = jnp.where(jnp.isfinite(mx), mx, 0.0)
        feats.append(mean_g)
        feats.append(max_g)
        if l < L - 1:
            h, keep = _pool(h, pool_w[l], seq, n_alive, src, dst, starts)
            n_alive = keep
            flag = jnp.where(keep, 0.0, NEG).astype(jnp.float32)
    ga = jnp.concatenate(feats, axis=1)
    return _head(ga, gru_wih, gru_whh, gru_bih, gru_bhh, lin1_w, lin1_b, lin2_w, lin2_b)
